# baseline (speedup 1.0000x reference)
import jax
import jax.numpy as jnp
from jax import lax
from jax.experimental import pallas as pl
from jax.experimental.pallas import tpu as pltpu
from jax.experimental.pallas import tpu_sc as plsc

_NUM_TOKENS = 32768
_TOP_K = 8
_LANES = 16
_NUM_WORKERS = 32
_TOKS_PW = _NUM_TOKENS // _NUM_WORKERS

_SORT_NET = (
    (0, 1), (2, 3), (0, 2), (1, 3), (1, 2),
    (4, 5), (6, 7), (4, 6), (5, 7), (5, 6),
    (0, 4), (1, 5), (2, 6), (3, 7),
    (2, 4), (3, 5),
    (1, 2), (3, 4), (5, 6),
)


def _sc_body(idx_hbm, sc_hbm, out_s_hbm, out_i_hbm, idx_v, sc_v, os_v, oi_v):
    wid = lax.axis_index("s") * 2 + lax.axis_index("c")
    tbase = wid * _TOKS_PW
    pltpu.sync_copy(idx_hbm.at[:, pl.ds(tbase, _TOKS_PW)], idx_v)
    pltpu.sync_copy(sc_hbm.at[:, pl.ds(tbase, _TOKS_PW)], sc_v)

    @plsc.parallel_loop(0, _TOKS_PW // _LANES, unroll=8)
    def body(g):
        b = g * _LANES
        comp = [idx_v[k, pl.ds(b, _LANES)] * 8 + k for k in range(_TOP_K)]
        scs = [sc_v[k, pl.ds(b, _LANES)] for k in range(_TOP_K)]
        for i, j in _SORT_NET:
            lt = comp[i] < comp[j]
            comp[i], comp[j] = (
                jnp.where(lt, comp[i], comp[j]),
                jnp.where(lt, comp[j], comp[i]),
            )
            scs[i], scs[j] = (
                jnp.where(lt, scs[i], scs[j]),
                jnp.where(lt, scs[j], scs[i]),
            )
        for r in range(_TOP_K):
            os_v[r, pl.ds(b, _LANES)] = scs[r]
            oi_v[r, pl.ds(b, _LANES)] = (comp[r] & 7) >> 3

    pltpu.sync_copy(os_v, out_s_hbm.at[:, pl.ds(tbase, _TOKS_PW)])
    pltpu.sync_copy(oi_v, out_i_hbm.at[:, pl.ds(tbase, _TOKS_PW)])


def kernel(top_scores, selected_experts_indices):
    run = pl.kernel(
        _sc_body,
        out_type=(
            jax.ShapeDtypeStruct((_TOP_K, _NUM_TOKENS), jnp.float32),
            jax.ShapeDtypeStruct((_TOP_K, _NUM_TOKENS), jnp.int32),
        ),
        mesh=plsc.VectorSubcoreMesh(core_axis_name="c", subcore_axis_name="s"),
        compiler_params=pltpu.CompilerParams(
            needs_layout_passes=False,
            use_tc_tiling_on_sc=True,
            skip_device_barrier=True,
        ),
        scratch_types=[
            pltpu.VMEM((_TOP_K, _TOKS_PW), jnp.int32),
            pltpu.VMEM((_TOP_K, _TOKS_PW), jnp.float32),
            pltpu.VMEM((_TOP_K, _TOKS_PW), jnp.float32),
            pltpu.VMEM((_TOP_K, _TOKS_PW), jnp.int32),
        ],
    )
    os_t, oi_t = run(selected_experts_indices.T, top_scores.T)
    return os_t.T, oi_t.T


# speedup vs baseline: 1.0372x; 1.0372x over previous
import jax
import jax.numpy as jnp
from jax import lax
from jax.experimental import pallas as pl
from jax.experimental.pallas import tpu as pltpu
from jax.experimental.pallas import tpu_sc as plsc

_NUM_TOKENS = 32768
_TOP_K = 8
_LANES = 16
_NUM_WORKERS = 32
_TOKS_PW = _NUM_TOKENS // _NUM_WORKERS

_SORT_NET = (
    (0, 1), (2, 3), (0, 2), (1, 3), (1, 2),
    (4, 5), (6, 7), (4, 6), (5, 7), (5, 6),
    (0, 4), (1, 5), (2, 6), (3, 7),
    (2, 4), (3, 5),
    (1, 2), (3, 4), (5, 6),
)


_HALF = _TOKS_PW // 2


def _sc_body(idx_hbm, sc_hbm, out_s_hbm, out_i_hbm,
             idx_v, sc_v, os_v, oi_v, in_sems, out_sems):
    wid = lax.axis_index("s") * 2 + lax.axis_index("c")
    tbase = wid * _TOKS_PW

    def start_in(h):
        hb = tbase + h * _HALF
        ci = pltpu.async_copy(
            idx_hbm.at[:, pl.ds(hb, _HALF)], idx_v.at[h], in_sems.at[h])
        cs = pltpu.async_copy(
            sc_hbm.at[:, pl.ds(hb, _HALF)], sc_v.at[h], in_sems.at[h])
        return ci, cs

    def compute(h):
        @plsc.parallel_loop(0, _HALF // _LANES, unroll=4)
        def body(g):
            b = g * _LANES
            comp = [idx_v[h, k, pl.ds(b, _LANES)] * 8 + k
                    for k in range(_TOP_K)]
            scs = [sc_v[h, k, pl.ds(b, _LANES)] for k in range(_TOP_K)]
            for i, j in _SORT_NET:
                lt = comp[i] < comp[j]
                comp[i], comp[j] = (
                    jnp.where(lt, comp[i], comp[j]),
                    jnp.where(lt, comp[j], comp[i]),
                )
                scs[i], scs[j] = (
                    jnp.where(lt, scs[i], scs[j]),
                    jnp.where(lt, scs[j], scs[i]),
                )
            for r in range(_TOP_K):
                os_v[h, r, pl.ds(b, _LANES)] = scs[r]
                oi_v[h, r, pl.ds(b, _LANES)] = (comp[r] & 7) >> 3

    def start_out(h):
        hb = tbase + h * _HALF
        co = pltpu.async_copy(
            os_v.at[h], out_s_hbm.at[:, pl.ds(hb, _HALF)], out_sems.at[h])
        cz = pltpu.async_copy(
            oi_v.at[h], out_i_hbm.at[:, pl.ds(hb, _HALF)], out_sems.at[h])
        return co, cz

    i0a, i0b = start_in(0)
    i1a, i1b = start_in(1)
    i0a.wait()
    i0b.wait()
    compute(0)
    o0a, o0b = start_out(0)
    i1a.wait()
    i1b.wait()
    compute(1)
    o1a, o1b = start_out(1)
    o0a.wait()
    o0b.wait()
    o1a.wait()
    o1b.wait()


def kernel(top_scores, selected_experts_indices):
    run = pl.kernel(
        _sc_body,
        out_type=(
            jax.ShapeDtypeStruct((_TOP_K, _NUM_TOKENS), jnp.float32),
            jax.ShapeDtypeStruct((_TOP_K, _NUM_TOKENS), jnp.int32),
        ),
        mesh=plsc.VectorSubcoreMesh(core_axis_name="c", subcore_axis_name="s"),
        compiler_params=pltpu.CompilerParams(
            needs_layout_passes=False,
            use_tc_tiling_on_sc=True,
            skip_device_barrier=True,
        ),
        scratch_types=[
            pltpu.VMEM((2, _TOP_K, _HALF), jnp.int32),
            pltpu.VMEM((2, _TOP_K, _HALF), jnp.float32),
            pltpu.VMEM((2, _TOP_K, _HALF), jnp.float32),
            pltpu.VMEM((2, _TOP_K, _HALF), jnp.int32),
            pltpu.SemaphoreType.DMA((2,)),
            pltpu.SemaphoreType.DMA((2,)),
        ],
    )
    os_t, oi_t = run(selected_experts_indices.T, top_scores.T)
    return os_t.T, oi_t.T
